# baseline (speedup 1.0000x reference)
import jax
import jax.numpy as jnp
from jax import lax
from jax.experimental import pallas as pl
from jax.experimental.pallas import tpu as pltpu
from jax.experimental.pallas import tpu_sc as plsc

NC = 2
NS = 16
NW = NC * NS
L = 16

V = 200
B = 4096 * 200
D = 256
B_PER_W = B // NW
CHUNK = 64
N_CHUNKS = B_PER_W // CHUNK
GROUPS = CHUNK // L
COLS = D // L
IDX_ROWS = B_PER_W // 128


def _gather_body(idx_hbm, table_hbm, out_hbm, table_v, idx_v, r0, r1, o0, o1):
    rows = (r0, r1)
    osem = (o0, o1)
    wid = lax.axis_index("s") * NC + lax.axis_index("c")
    base = wid * B_PER_W
    pltpu.sync_copy(table_hbm, table_v)
    pltpu.sync_copy(idx_hbm.at[wid], idx_v)

    def pair(kk, carry):
        for b in range(2):
            k = 2 * kk + b

            @pl.when(kk > 0)
            def _drain():
                pltpu.make_async_copy(
                    rows[b],
                    out_hbm.at[pl.ds(base + (k - 2) * CHUNK, CHUNK)],
                    osem[b],
                ).wait()

            @plsc.parallel_loop(0, CHUNK, step=L, unroll=4)
            def grp(j0):
                iv = idx_v[kk, pl.ds(b * CHUNK + j0, L)]
                prev = None
                for l in range(L):
                    t = iv[l]
                    cur = []
                    for c in range(COLS):
                        cur.append(table_v[t, pl.ds(L * c, L)])
                        if prev is not None:
                            rows[b][j0 + l - 1, pl.ds(L * c, L)] = prev[c]
                    prev = cur
                for c in range(COLS):
                    rows[b][j0 + L - 1, pl.ds(L * c, L)] = prev[c]
            pltpu.async_copy(
                rows[b], out_hbm.at[pl.ds(base + k * CHUNK, CHUNK)], osem[b]
            )
        return carry

    lax.fori_loop(0, N_CHUNKS // 2, pair, 0)
    for b in range(2):
        k = N_CHUNKS - 2 + b
        pltpu.make_async_copy(
            rows[b], out_hbm.at[pl.ds(base + k * CHUNK, CHUNK)], osem[b]
        ).wait()


def kernel(frameIndices, numFrames, frameEmbed_weight):
    del numFrames
    idx = frameIndices.astype(jnp.int32).reshape(NW, IDX_ROWS, 128)
    mesh = plsc.VectorSubcoreMesh(
        core_axis_name="c", subcore_axis_name="s", num_cores=NC, num_subcores=NS
    )
    out = pl.kernel(
        _gather_body,
        out_type=jax.ShapeDtypeStruct((B, D), jnp.float32),
        mesh=mesh,
        compiler_params=pltpu.CompilerParams(needs_layout_passes=False),
        scratch_types=(
            [
                pltpu.VMEM((V, D), jnp.float32),
                pltpu.VMEM((IDX_ROWS, 128), jnp.int32),
                pltpu.VMEM((CHUNK, D), jnp.float32),
                pltpu.VMEM((CHUNK, D), jnp.float32),
            ]
            + [pltpu.SemaphoreType.DMA for _ in range(2)]
        ),
    )(idx, frameEmbed_weight)
    return out.reshape(frameIndices.shape[0], frameIndices.shape[1], D)


# speedup vs baseline: 2.0904x; 2.0904x over previous
import jax
import jax.numpy as jnp
from jax import lax
from jax.experimental import pallas as pl
from jax.experimental.pallas import tpu as pltpu
from jax.experimental.pallas import tpu_sc as plsc

NC = 2
NS = 16
NW = NC * NS
L = 16

V = 200
B = 4096 * 200
D = 256
B_PER_W = B // NW
CHUNK = 64
N_CHUNKS = B_PER_W // CHUNK
GROUPS = CHUNK // L
COLS = D // L
IDX_ROWS = B_PER_W // 128


def _gather_body(idx_hbm, table_hbm, out_hbm, table_v, idx_v, r0, r1, o0, o1):
    rows = (r0, r1)
    osem = (o0, o1)
    wid = lax.axis_index("s") * NC + lax.axis_index("c")
    base = wid * B_PER_W
    pltpu.sync_copy(table_hbm, table_v)
    pltpu.sync_copy(idx_hbm.at[wid], idx_v)

    def pair(kk, carry):
        for b in range(2):
            k = 2 * kk + b

            @pl.when(kk > 0)
            def _drain():
                pltpu.make_async_copy(
                    rows[b],
                    out_hbm.at[pl.ds(base + (k - 2) * CHUNK, CHUNK)],
                    osem[b],
                ).wait()

            @plsc.parallel_loop(0, CHUNK, step=L, unroll=1)
            def grp(j0):
                iv = idx_v[kk, pl.ds(b * CHUNK + j0, L)]
                prev = None
                for l in range(L):
                    t = iv[l]
                    cur = []
                    for c in range(COLS):
                        cur.append(table_v[t, pl.ds(L * c, L)])
                        if prev is not None:
                            rows[b][j0 + l - 1, pl.ds(L * c, L)] = prev[c]
                    prev = cur
                for c in range(COLS):
                    rows[b][j0 + L - 1, pl.ds(L * c, L)] = prev[c]
            pltpu.async_copy(
                rows[b], out_hbm.at[pl.ds(base + k * CHUNK, CHUNK)], osem[b]
            )
        return carry

    lax.fori_loop(0, N_CHUNKS // 2, pair, 0)
    for b in range(2):
        k = N_CHUNKS - 2 + b
        pltpu.make_async_copy(
            rows[b], out_hbm.at[pl.ds(base + k * CHUNK, CHUNK)], osem[b]
        ).wait()


def kernel(frameIndices, numFrames, frameEmbed_weight):
    del numFrames
    idx = frameIndices.astype(jnp.int32).reshape(NW, IDX_ROWS, 128)
    mesh = plsc.VectorSubcoreMesh(
        core_axis_name="c", subcore_axis_name="s", num_cores=NC, num_subcores=NS
    )
    out = pl.kernel(
        _gather_body,
        out_type=jax.ShapeDtypeStruct((B, D), jnp.float32),
        mesh=mesh,
        compiler_params=pltpu.CompilerParams(needs_layout_passes=False),
        scratch_types=(
            [
                pltpu.VMEM((V, D), jnp.float32),
                pltpu.VMEM((IDX_ROWS, 128), jnp.int32),
                pltpu.VMEM((CHUNK, D), jnp.float32),
                pltpu.VMEM((CHUNK, D), jnp.float32),
            ]
            + [pltpu.SemaphoreType.DMA for _ in range(2)]
        ),
    )(idx, frameEmbed_weight)
    return out.reshape(frameIndices.shape[0], frameIndices.shape[1], D)


# speedup vs baseline: 2.2465x; 1.0747x over previous
import jax
import jax.numpy as jnp
from jax import lax
from jax.experimental import pallas as pl
from jax.experimental.pallas import tpu as pltpu
from jax.experimental.pallas import tpu_sc as plsc

NC = 2
NS = 16
NW = NC * NS
L = 16

V = 200
B = 4096 * 200
D = 256
B_PER_W = B // NW
CHUNK = 64
N_CHUNKS = B_PER_W // CHUNK
GROUPS = CHUNK // L
COLS = D // L
IDX_ROWS = B_PER_W // 128


def _gather_body(idx_hbm, table_hbm, out_hbm, table_f, idx_v, r0, r1, o0, o1):
    rows = (r0, r1)
    osem = (o0, o1)
    wid = lax.axis_index("s") * NC + lax.axis_index("c")
    base = wid * B_PER_W
    pltpu.sync_copy(table_hbm, table_f)
    pltpu.sync_copy(idx_hbm.at[wid], idx_v)

    def pair(kk, carry):
        for b in range(2):
            k = 2 * kk + b

            @pl.when(kk > 0)
            def _drain():
                pltpu.make_async_copy(
                    rows[b],
                    out_hbm.at[pl.ds(base + (k - 2) * CHUNK, CHUNK)],
                    osem[b],
                ).wait()

            @plsc.parallel_loop(0, CHUNK, step=L, unroll=2)
            def grp(j0):
                iv = idx_v[kk, pl.ds(b * CHUNK + j0, L)]
                prev = None
                for l in range(L):
                    t = iv[l]
                    cur = []
                    for c in range(COLS):
                        cur.append(table_f[pl.ds(t + L * c, L)])
                        if prev is not None:
                            rows[b][j0 + l - 1, pl.ds(L * c, L)] = prev[c]
                    prev = cur
                for c in range(COLS):
                    rows[b][j0 + L - 1, pl.ds(L * c, L)] = prev[c]
            pltpu.async_copy(
                rows[b], out_hbm.at[pl.ds(base + k * CHUNK, CHUNK)], osem[b]
            )
        return carry

    lax.fori_loop(0, N_CHUNKS // 2, pair, 0)
    for b in range(2):
        k = N_CHUNKS - 2 + b
        pltpu.make_async_copy(
            rows[b], out_hbm.at[pl.ds(base + k * CHUNK, CHUNK)], osem[b]
        ).wait()


def kernel(frameIndices, numFrames, frameEmbed_weight):
    del numFrames
    idx = (frameIndices.astype(jnp.int32) * D).reshape(NW, IDX_ROWS, 128)
    mesh = plsc.VectorSubcoreMesh(
        core_axis_name="c", subcore_axis_name="s", num_cores=NC, num_subcores=NS
    )
    out = pl.kernel(
        _gather_body,
        out_type=jax.ShapeDtypeStruct((B, D), jnp.float32),
        mesh=mesh,
        compiler_params=pltpu.CompilerParams(needs_layout_passes=False),
        scratch_types=(
            [
                pltpu.VMEM((V * D,), jnp.float32),
                pltpu.VMEM((IDX_ROWS, 128), jnp.int32),
                pltpu.VMEM((CHUNK, D), jnp.float32),
                pltpu.VMEM((CHUNK, D), jnp.float32),
            ]
            + [pltpu.SemaphoreType.DMA for _ in range(2)]
        ),
    )(idx, frameEmbed_weight.reshape(V * D))
    return out.reshape(frameIndices.shape[0], frameIndices.shape[1], D)
